# Initial kernel scaffold; baseline (speedup 1.0000x reference)
import functools
import math

import jax
import jax.numpy as jnp
from jax import lax
from jax.experimental import pallas as pl
from jax.experimental.pallas import tpu as pltpu
from jax.experimental.pallas import tpu_sc as plsc

B, L, V, D, P = 4096, 200, 100000, 128, 512
N = B * L
NW = 32
TOK_PER_W = N // NW
CHUNK = 128
N_CHUNKS = TOK_PER_W // CHUNK
GROUPS = CHUNK // 16
EPS = 1e-5 / D
U = 8


def _rsqrt(v):
    bits = plsc.bitcast(v, jnp.int32)
    y = plsc.bitcast(jnp.int32(0x5F3759DF) - (bits >> 1), jnp.float32)
    for _ in range(3):
        y = y * (1.5 - 0.5 * v * y * y)
    return y


_mesh = plsc.VectorSubcoreMesh(core_axis_name="c", subcore_axis_name="s")


@functools.partial(
    pl.kernel,
    mesh=_mesh,
    out_type=jax.ShapeDtypeStruct((N, D), jnp.float32),
    scratch_types=[
        pltpu.VMEM((P, D), jnp.float32),
        pltpu.VMEM((CHUNK,), jnp.int32),
        pltpu.VMEM((CHUNK,), jnp.int32),
        pltpu.VMEM((CHUNK, D), jnp.float32),
        pltpu.VMEM((D, 16), jnp.float32),
        pltpu.VMEM((D,), jnp.float32),
        pltpu.VMEM((D,), jnp.float32),
        pltpu.SemaphoreType.DMA,
    ],
)
def _embed_ln(ids_hbm, ord_hbm, wtab_hbm, pe_hbm, gamma_hbm, beta_hbm,
              out_hbm, pe_v, ids_v, ord_v, x_v, t_v, g_v, b_v, sem):
    wid = lax.axis_index("s") * 2 + lax.axis_index("c")
    base0 = wid * TOK_PER_W
    pltpu.sync_copy(pe_hbm, pe_v)
    pltpu.sync_copy(gamma_hbm, g_v)
    pltpu.sync_copy(beta_hbm, b_v)

    def chunk_body(ci, _):
        base = base0 + ci * CHUNK
        pltpu.sync_copy(ids_hbm.at[pl.ds(base, CHUNK)], ids_v)
        pltpu.sync_copy(ord_hbm.at[pl.ds(base, CHUNK)], ord_v)
        pltpu.async_copy(wtab_hbm.at[ids_v], x_v, sem).wait()
        for g in range(GROUPS):
            row16 = jnp.int32(g * 16) + lax.iota(jnp.int32, 16)
            ord16 = ord_v[pl.ds(g * 16, 16)]
            zero = jnp.zeros((16,), jnp.float32)

            def p1(i, carry):
                s, q = carry
                d0 = i * U
                for j in range(U):
                    dvec = jnp.full((16,), d0 + j, jnp.int32)
                    xw = plsc.load_gather(x_v, [row16, dvec])
                    xp = plsc.load_gather(pe_v, [ord16, dvec])
                    x = xw + xp
                    t_v[d0 + j, :] = x
                    s = s + x
                    q = q + x * x
                return (s, q)

            s, q = lax.fori_loop(0, D // U, p1, (zero, zero))
            mean = s * (1.0 / D)
            var = q * (1.0 / D) - mean * mean
            rstd = _rsqrt(var + EPS)

            def p2(i, c):
                d0 = i * U
                for j in range(U):
                    x = t_v[d0 + j, :]
                    gd = g_v[d0 + j]
                    bd = b_v[d0 + j]
                    y = (x - mean) * rstd * gd + bd
                    dvec = jnp.full((16,), d0 + j, jnp.int32)
                    plsc.store_scatter(x_v, [row16, dvec], y)
                return c

            lax.fori_loop(0, D // U, p2, 0)
        pltpu.sync_copy(x_v, out_hbm.at[pl.ds(base, CHUNK)])
        return 0

    lax.fori_loop(0, N_CHUNKS, chunk_body, 0)


def kernel(masked_token_ids, visit_concept_orders, word_embeddings, pe,
           gamma, beta):
    ids = masked_token_ids.reshape(-1).astype(jnp.int32)
    orders = visit_concept_orders.reshape(-1).astype(jnp.int32)
    pe_s = (pe * (1.0 / math.sqrt(D))).astype(jnp.float32)
    out = _embed_ln(ids, orders, word_embeddings, pe_s,
                    gamma.astype(jnp.float32), beta.astype(jnp.float32))
    return out.reshape(masked_token_ids.shape[0], masked_token_ids.shape[1], D)


# speedup vs baseline: 4.3850x; 4.3850x over previous
import functools
import math

import jax
import jax.numpy as jnp
from jax import lax
from jax.experimental import pallas as pl
from jax.experimental.pallas import tpu as pltpu
from jax.experimental.pallas import tpu_sc as plsc

B, L, V, D, P = 4096, 200, 100000, 128, 512
N = B * L
NW = 32
TOK_PER_W = N // NW
CHUNK = 128
N_CHUNKS = TOK_PER_W // CHUNK
K = D // 16
EPS = 1e-5 / D
U = 2


def _rsqrt(v):
    bits = plsc.bitcast(v, jnp.int32)
    y = plsc.bitcast(jnp.int32(0x5F3759DF) - (bits >> 1), jnp.float32)
    for _ in range(3):
        y = y * (1.5 - 0.5 * v * y * y)
    return y


_mesh = plsc.VectorSubcoreMesh(core_axis_name="c", subcore_axis_name="s")


@functools.partial(
    pl.kernel,
    mesh=_mesh,
    out_type=jax.ShapeDtypeStruct((N, D), jnp.float32),
    compiler_params=pltpu.CompilerParams(needs_layout_passes=False),
    scratch_types=[
        pltpu.VMEM((CHUNK,), jnp.int32),
        pltpu.VMEM((CHUNK,), jnp.int32),
        pltpu.VMEM((CHUNK, D), jnp.float32),
        pltpu.VMEM((CHUNK, D), jnp.float32),
        pltpu.VMEM((D,), jnp.float32),
        pltpu.VMEM((D,), jnp.float32),
        pltpu.SemaphoreType.DMA,
        pltpu.SemaphoreType.DMA,
    ],
)
def _embed_ln(ids_hbm, ord_hbm, wtab_hbm, pe_hbm, gamma_hbm, beta_hbm,
              out_hbm, ids_v, ord_v, x_v, p_v, g_v, b_v, sem1, sem2):
    wid = lax.axis_index("s") * 2 + lax.axis_index("c")
    base0 = wid * TOK_PER_W
    pltpu.sync_copy(gamma_hbm, g_v)
    pltpu.sync_copy(beta_hbm, b_v)
    gk = [g_v[pl.ds(k * 16, 16)] for k in range(K)]
    bk = [b_v[pl.ds(k * 16, 16)] for k in range(K)]

    def chunk_body(ci, _):
        base = base0 + ci * CHUNK
        pltpu.sync_copy(ids_hbm.at[pl.ds(base, CHUNK)], ids_v)
        pltpu.sync_copy(ord_hbm.at[pl.ds(base, CHUNK)], ord_v)
        cp1 = pltpu.async_copy(wtab_hbm.at[ids_v], x_v, sem1)
        cp2 = pltpu.async_copy(pe_hbm.at[ord_v], p_v, sem2)
        cp1.wait()
        cp2.wait()

        def tok(i, c):
            for j in range(U):
                t = i * U + j
                x = [
                    x_v[t, pl.ds(k * 16, 16)] + p_v[t, pl.ds(k * 16, 16)]
                    for k in range(K)
                ]
                s = x[0]
                q = x[0] * x[0]
                for k in range(1, K):
                    s = s + x[k]
                    q = q + x[k] * x[k]
                tot = jnp.sum(s)
                totq = jnp.sum(q)
                mean = tot * (1.0 / D)
                var = totq * (1.0 / D) - mean * mean
                mean16 = jnp.full((16,), mean, jnp.float32)
                rstd16 = _rsqrt(jnp.full((16,), var + EPS, jnp.float32))
                for k in range(K):
                    y = (x[k] - mean16) * rstd16 * gk[k] + bk[k]
                    x_v[t, pl.ds(k * 16, 16)] = y
            return c

        lax.fori_loop(0, CHUNK // U, tok, 0)
        pltpu.sync_copy(x_v, out_hbm.at[pl.ds(base, CHUNK)])
        return 0

    lax.fori_loop(0, N_CHUNKS, chunk_body, 0)


def kernel(masked_token_ids, visit_concept_orders, word_embeddings, pe,
           gamma, beta):
    ids = masked_token_ids.reshape(-1).astype(jnp.int32)
    orders = visit_concept_orders.reshape(-1).astype(jnp.int32)
    pe_s = (pe * (1.0 / math.sqrt(D))).astype(jnp.float32)
    out = _embed_ln(ids, orders, word_embeddings, pe_s,
                    gamma.astype(jnp.float32), beta.astype(jnp.float32))
    return out.reshape(masked_token_ids.shape[0], masked_token_ids.shape[1], D)


# speedup vs baseline: 8.3332x; 1.9004x over previous
import functools
import math

import jax
import jax.numpy as jnp
from jax import lax
from jax.experimental import pallas as pl
from jax.experimental.pallas import tpu as pltpu
from jax.experimental.pallas import tpu_sc as plsc

B, L, V, D, P = 4096, 200, 100000, 128, 512
N = B * L
NW = 32
TOK_PER_W = N // NW
CHUNK = 128
N_CHUNKS = TOK_PER_W // CHUNK
K = D // 16
EPS = 1e-5 / D
U = 2


def _rsqrt(v):
    bits = plsc.bitcast(v, jnp.int32)
    y = plsc.bitcast(jnp.int32(0x5F3759DF) - (bits >> 1), jnp.float32)
    for _ in range(3):
        y = y * (1.5 - 0.5 * v * y * y)
    return y


_mesh = plsc.VectorSubcoreMesh(core_axis_name="c", subcore_axis_name="s")


@functools.partial(
    pl.kernel,
    mesh=_mesh,
    out_type=jax.ShapeDtypeStruct((N, D), jnp.float32),
    compiler_params=pltpu.CompilerParams(needs_layout_passes=False),
    scratch_types=[
        pltpu.VMEM((2, CHUNK), jnp.int32),
        pltpu.VMEM((2, CHUNK), jnp.int32),
        pltpu.VMEM((2, CHUNK, D), jnp.float32),
        pltpu.VMEM((2, CHUNK, D), jnp.float32),
        pltpu.VMEM((2, CHUNK, D), jnp.float32),
        pltpu.VMEM((D,), jnp.float32),
        pltpu.VMEM((D,), jnp.float32),
        pltpu.SemaphoreType.DMA,
        pltpu.SemaphoreType.DMA,
        pltpu.SemaphoreType.DMA,
        pltpu.SemaphoreType.DMA,
        pltpu.SemaphoreType.DMA,
        pltpu.SemaphoreType.DMA,
    ],
)
def _embed_ln(ids_hbm, ord_hbm, wtab_hbm, pe_hbm, gamma_hbm, beta_hbm,
              out_hbm, ids_v, ord_v, x_v, p_v, o_v, g_v, b_v,
              si0, si1, sg0, sg1, so0, so1):
    sem_i = (si0, si1)
    sem_g = (sg0, sg1)
    sem_o = (so0, so1)
    wid = lax.axis_index("s") * 2 + lax.axis_index("c")
    base0 = wid * TOK_PER_W
    pltpu.sync_copy(gamma_hbm, g_v)
    pltpu.sync_copy(beta_hbm, b_v)
    gk = [g_v[pl.ds(k * 16, 16)] for k in range(K)]
    bk = [b_v[pl.ds(k * 16, 16)] for k in range(K)]

    def issue_ids(ci, p):
        base = base0 + ci * CHUNK
        pltpu.async_copy(ids_hbm.at[pl.ds(base, CHUNK)], ids_v.at[p], sem_i[p])
        pltpu.async_copy(ord_hbm.at[pl.ds(base, CHUNK)], ord_v.at[p], sem_i[p])

    def wait_ids(p):
        pltpu.make_async_copy(ids_hbm.at[pl.ds(0, CHUNK)], ids_v.at[p],
                              sem_i[p]).wait()
        pltpu.make_async_copy(ord_hbm.at[pl.ds(0, CHUNK)], ord_v.at[p],
                              sem_i[p]).wait()

    def issue_gathers(p):
        pltpu.async_copy(wtab_hbm.at[ids_v.at[p]], x_v.at[p], sem_g[p])
        pltpu.async_copy(pe_hbm.at[ord_v.at[p]], p_v.at[p], sem_g[p])

    def wait_gathers(p):
        pltpu.make_async_copy(wtab_hbm.at[ids_v.at[p]], x_v.at[p],
                              sem_g[p]).wait()
        pltpu.make_async_copy(pe_hbm.at[ord_v.at[p]], p_v.at[p],
                              sem_g[p]).wait()

    def wait_out(p):
        pltpu.make_async_copy(o_v.at[p], out_hbm.at[pl.ds(0, CHUNK)],
                              sem_o[p]).wait()

    issue_ids(0, 0)
    issue_ids(1, 1)
    wait_ids(0)
    issue_gathers(0)

    def outer(ii, _):
        for p in range(2):
            ci = 2 * ii + p
            wait_gathers(p)

            @pl.when(ci + 2 < N_CHUNKS)
            def _():
                issue_ids(ci + 2, p)

            @pl.when(ci + 1 < N_CHUNKS)
            def _():
                wait_ids(1 - p)
                issue_gathers(1 - p)

            @pl.when(ci >= 2)
            def _():
                wait_out(p)

            def tok(i, c):
                for j in range(U):
                    t = i * U + j
                    x = [
                        x_v[p, t, pl.ds(k * 16, 16)]
                        + p_v[p, t, pl.ds(k * 16, 16)]
                        for k in range(K)
                    ]
                    s = x[0]
                    q = x[0] * x[0]
                    for k in range(1, K):
                        s = s + x[k]
                        q = q + x[k] * x[k]
                    tot = jnp.sum(s)
                    totq = jnp.sum(q)
                    mean = tot * (1.0 / D)
                    var = totq * (1.0 / D) - mean * mean
                    mean16 = jnp.full((16,), mean, jnp.float32)
                    rstd16 = _rsqrt(jnp.full((16,), var + EPS, jnp.float32))
                    for k in range(K):
                        y = (x[k] - mean16) * rstd16 * gk[k] + bk[k]
                        o_v[p, t, pl.ds(k * 16, 16)] = y
                return c

            lax.fori_loop(0, CHUNK // U, tok, 0)
            base = base0 + ci * CHUNK
            pltpu.async_copy(o_v.at[p], out_hbm.at[pl.ds(base, CHUNK)],
                             sem_o[p])
        return 0

    lax.fori_loop(0, N_CHUNKS // 2, outer, 0)
    wait_out(0)
    wait_out(1)


def kernel(masked_token_ids, visit_concept_orders, word_embeddings, pe,
           gamma, beta):
    ids = masked_token_ids.reshape(-1).astype(jnp.int32)
    orders = visit_concept_orders.reshape(-1).astype(jnp.int32)
    pe_s = (pe * (1.0 / math.sqrt(D))).astype(jnp.float32)
    out = _embed_ln(ids, orders, word_embeddings, pe_s,
                    gamma.astype(jnp.float32), beta.astype(jnp.float32))
    return out.reshape(masked_token_ids.shape[0], masked_token_ids.shape[1], D)
